# baseline (speedup 1.0000x reference)
import functools

import jax
import jax.numpy as jnp
from jax import lax
from jax.experimental import pallas as pl
from jax.experimental.pallas import tpu as pltpu
from jax.experimental.pallas import tpu_sc as plsc

NUM_SEGMENTS = 256
E = 320000
D = 128

NC = 2
NS = 16
NW = NC * NS
ROWS_PER_TILE = E // NW
BLK = 128
NFULL = ROWS_PER_TILE // BLK
TAIL = ROWS_PER_TILE - NFULL * BLK
IDROWS = NFULL + 2
SEGS_PER_TILE = NUM_SEGMENTS // NS
NSLOT = 4
NMAIN = (NFULL // NSLOT) * NSLOT

_mesh = plsc.VectorSubcoreMesh(core_axis_name="c", subcore_axis_name="s")


def _seg_sum_body(feat, ids2, out, fbuf, tbuf, ibuf, zbuf, acc,
                  gs0, gs1, gs2, gs3, ss0, ss1, ss2, ss3, semi):
    c = lax.axis_index("c")
    s = lax.axis_index("s")
    gsems = (gs0, gs1, gs2, gs3)
    ssems = (ss0, ss1, ss2, ss3)
    w = s * NC + c
    base = w * ROWS_PER_TILE

    pltpu.async_copy(ids2.at[pl.ds(w * IDROWS, IDROWS)], ibuf, semi)
    pltpu.async_copy(
        feat.at[pl.ds(base + NFULL * BLK, TAIL), :],
        tbuf.at[pl.ds(0, TAIL)], semi)

    zero16 = jnp.zeros((16,), jnp.float32)
    for r in range(SEGS_PER_TILE):
        for j in range(D // 16):
            zbuf[r, pl.ds(j * 16, 16)] = zero16
    for r in range(TAIL, BLK):
        for j in range(D // 16):
            tbuf[r, pl.ds(j * 16, 16)] = zero16

    pltpu.sync_copy(zbuf, acc.at[pl.ds(s * SEGS_PER_TILE, SEGS_PER_TILE)])
    plsc.subcore_barrier()

    def start_gather(i, b):
        pltpu.async_copy(
            feat.at[pl.ds(base + i * BLK, BLK), :], fbuf.at[b], gsems[b])

    def wait_gather(b):
        pltpu.make_async_copy(
            feat.at[pl.ds(0, BLK), :], fbuf.at[b], gsems[b]).wait()

    def start_scatter(i, b):
        pltpu.async_copy(fbuf.at[b], acc.at[ibuf.at[i]], ssems[b], add=True)

    def wait_scatter(b):
        pltpu.make_async_copy(
            fbuf.at[b], acc.at[ibuf.at[0]], ssems[b]).wait()

    for b in range(NSLOT):
        start_gather(b, b)

    pltpu.make_async_copy(
        ids2.at[pl.ds(0, IDROWS)], ibuf, semi).wait()
    pltpu.make_async_copy(
        feat.at[pl.ds(0, TAIL), :], tbuf.at[pl.ds(0, TAIL)], semi).wait()

    def loop_body(iv, carry):
        for b in range(NSLOT):
            i = NSLOT * iv + b
            wait_gather(b)
            start_scatter(i, b)
            g = i + 2
            bg = (b + 2) % NSLOT

            @pl.when(g >= NSLOT)
            def _prefetch():
                wait_scatter(bg)
                start_gather(g, bg)
        return carry

    lax.fori_loop(0, NMAIN // NSLOT, loop_body, None)

    for i in range(NMAIN, NFULL):
        b = i % NSLOT
        wait_gather(b)
        start_scatter(i, b)
    for i in range(NFULL - NSLOT, NFULL):
        wait_scatter(i % NSLOT)

    pltpu.sync_copy(tbuf, acc.at[ibuf.at[NFULL]], add=True)

    plsc.subcore_barrier()
    seg0 = s * SEGS_PER_TILE
    pltpu.sync_copy(
        acc.at[pl.ds(seg0, SEGS_PER_TILE)],
        out.at[c, pl.ds(seg0, SEGS_PER_TILE), :])


_seg_sum = pl.kernel(
    _seg_sum_body,
    out_type=jax.ShapeDtypeStruct((NC, NUM_SEGMENTS, D), jnp.float32),
    mesh=_mesh,
    scratch_types=[
        pltpu.VMEM((NSLOT, BLK, D), jnp.float32),
        pltpu.VMEM((BLK, D), jnp.float32),
        pltpu.VMEM((IDROWS, BLK), jnp.int32),
        pltpu.VMEM((SEGS_PER_TILE, D), jnp.float32),
        pltpu.VMEM_SHARED((NUM_SEGMENTS, D), jnp.float32),
    ] + [pltpu.SemaphoreType.DMA] * 9,
)


def _combine_body(p_ref, o_ref):
    o_ref[...] = p_ref[0] + p_ref[1]


_combine = pl.pallas_call(
    _combine_body,
    out_shape=jax.ShapeDtypeStruct((NUM_SEGMENTS, D), jnp.float32),
)


def kernel(feat, segment_ids):
    ids2 = jnp.pad(
        segment_ids.reshape(NW, ROWS_PER_TILE),
        ((0, 0), (0, IDROWS * BLK - ROWS_PER_TILE)),
    ).reshape(NW * IDROWS, BLK)
    partials = _seg_sum(feat, ids2)
    return _combine(partials)


# speedup vs baseline: 1.1032x; 1.1032x over previous
import functools

import jax
import jax.numpy as jnp
from jax import lax
from jax.experimental import pallas as pl
from jax.experimental.pallas import tpu as pltpu
from jax.experimental.pallas import tpu_sc as plsc

NUM_SEGMENTS = 256
E = 320000
D = 128

NC = 2
NS = 16
NW = NC * NS
ROWS_PER_TILE = E // NW
BLK = 128
NFULL = ROWS_PER_TILE // BLK
TAIL = ROWS_PER_TILE - NFULL * BLK
IDROWS = NFULL + 2
SEGS_PER_TILE = NUM_SEGMENTS // NS

_mesh = plsc.VectorSubcoreMesh(core_axis_name="c", subcore_axis_name="s")


def _seg_sum_body(feat, ids2, out, fbuf, tbuf, ibuf, zbuf, mbuf, rbuf, acc,
                  sem0, sem1, semi):
    c = lax.axis_index("c")
    s = lax.axis_index("s")
    sems = (sem0, sem1)
    w = s * NC + c
    base = w * ROWS_PER_TILE
    reg0 = s * NUM_SEGMENTS

    pltpu.async_copy(ids2.at[pl.ds(w * IDROWS, IDROWS)], ibuf, semi)
    pltpu.async_copy(
        feat.at[pl.ds(base + NFULL * BLK, TAIL), :],
        tbuf.at[pl.ds(0, TAIL)], semi)

    zero16 = jnp.zeros((16,), jnp.float32)
    for r in range(SEGS_PER_TILE):
        for j in range(D // 16):
            zbuf[r, pl.ds(j * 16, 16)] = zero16
    for r in range(TAIL, BLK):
        for j in range(D // 16):
            tbuf[r, pl.ds(j * 16, 16)] = zero16

    for t in range(NUM_SEGMENTS // SEGS_PER_TILE):
        pltpu.sync_copy(
            zbuf, acc.at[pl.ds(reg0 + t * SEGS_PER_TILE, SEGS_PER_TILE)])

    pltpu.make_async_copy(
        ids2.at[pl.ds(0, IDROWS)], ibuf, semi).wait()
    pltpu.make_async_copy(
        feat.at[pl.ds(0, TAIL), :], tbuf.at[pl.ds(0, TAIL)], semi).wait()
    off16 = jnp.full((16,), reg0, jnp.int32)

    def bias_row(r, carry):
        for j in range(BLK // 16):
            ibuf[r, pl.ds(j * 16, 16)] = ibuf[r, pl.ds(j * 16, 16)] + off16
        return carry

    lax.fori_loop(0, IDROWS, bias_row, None)

    def start_block(i, b):
        pltpu.async_copy(
            feat.at[pl.ds(base + i * BLK, BLK), :], fbuf.at[b], sems[b])

    def wait_block(b):
        pltpu.make_async_copy(
            feat.at[pl.ds(0, BLK), :], fbuf.at[b], sems[b]).wait()

    start_block(0, 0)
    start_block(1, 1)

    def loop_body(iv, carry):
        for b in range(2):
            i = 2 * iv + b
            wait_block(b)
            pltpu.sync_copy(fbuf.at[b], acc.at[ibuf.at[i]], add=True)

            @pl.when(i + 2 < NFULL)
            def _prefetch():
                start_block(i + 2, b)
        return carry

    lax.fori_loop(0, NFULL // 2, loop_body, None)

    pltpu.sync_copy(tbuf, acc.at[ibuf.at[NFULL]], add=True)

    plsc.subcore_barrier()

    seg0 = s * SEGS_PER_TILE
    pltpu.sync_copy(acc.at[pl.ds(seg0, SEGS_PER_TILE)], mbuf)

    def merge_region(t, carry):
        pltpu.sync_copy(
            acc.at[pl.ds(t * NUM_SEGMENTS + seg0, SEGS_PER_TILE)], rbuf)
        for r in range(SEGS_PER_TILE):
            for j in range(D // 16):
                mbuf[r, pl.ds(j * 16, 16)] = (
                    mbuf[r, pl.ds(j * 16, 16)] + rbuf[r, pl.ds(j * 16, 16)])
        return carry

    lax.fori_loop(1, NS, merge_region, None)
    pltpu.sync_copy(mbuf, out.at[c, pl.ds(seg0, SEGS_PER_TILE), :])


_seg_sum = pl.kernel(
    _seg_sum_body,
    out_type=jax.ShapeDtypeStruct((NC, NUM_SEGMENTS, D), jnp.float32),
    mesh=_mesh,
    scratch_types=[
        pltpu.VMEM((2, BLK, D), jnp.float32),
        pltpu.VMEM((BLK, D), jnp.float32),
        pltpu.VMEM((IDROWS, BLK), jnp.int32),
        pltpu.VMEM((SEGS_PER_TILE, D), jnp.float32),
        pltpu.VMEM((SEGS_PER_TILE, D), jnp.float32),
        pltpu.VMEM((SEGS_PER_TILE, D), jnp.float32),
        pltpu.VMEM_SHARED((NS * NUM_SEGMENTS, D), jnp.float32),
        pltpu.SemaphoreType.DMA,
        pltpu.SemaphoreType.DMA,
        pltpu.SemaphoreType.DMA,
    ],
)


def _combine_body(p_ref, o_ref):
    o_ref[...] = p_ref[0] + p_ref[1]


_combine = pl.pallas_call(
    _combine_body,
    out_shape=jax.ShapeDtypeStruct((NUM_SEGMENTS, D), jnp.float32),
)


def kernel(feat, segment_ids):
    ids2 = jnp.pad(
        segment_ids.reshape(NW, ROWS_PER_TILE),
        ((0, 0), (0, IDROWS * BLK - ROWS_PER_TILE)),
    ).reshape(NW * IDROWS, BLK)
    partials = _seg_sum(feat, ids2)
    return _combine(partials)


# speedup vs baseline: 1.8124x; 1.6429x over previous
import functools

import jax
import jax.numpy as jnp
from jax import lax
from jax.experimental import pallas as pl
from jax.experimental.pallas import tpu as pltpu
from jax.experimental.pallas import tpu_sc as plsc

NUM_SEGMENTS = 256
E = 320000
D = 128

NC = 2
NS = 16
NW = NC * NS
ROWS_PER_TILE = E // NW
BLK = 128
NFULL = ROWS_PER_TILE // BLK
TAIL = ROWS_PER_TILE - NFULL * BLK
IDROWS = NFULL + 2
SEGS_PER_TILE = NUM_SEGMENTS // NS
RUNROLL = 4

_mesh = plsc.VectorSubcoreMesh(core_axis_name="c", subcore_axis_name="s")


def _seg_sum_body(feat, ids2, out, fbuf, tbuf, ibuf, iibuf, zbuf, pacc, acc,
                  sem0, sem1, semi):
    c = lax.axis_index("c")
    s = lax.axis_index("s")
    sems = (sem0, sem1)
    w = s * NC + c
    base = w * ROWS_PER_TILE

    pltpu.async_copy(ids2.at[pl.ds(w * IDROWS, IDROWS)], ibuf, semi)
    pltpu.async_copy(
        feat.at[pl.ds(base + NFULL * BLK, TAIL), :],
        tbuf.at[pl.ds(0, TAIL)], semi)

    zero16 = jnp.zeros((16,), jnp.float32)
    for r in range(SEGS_PER_TILE):
        for j in range(D // 16):
            zbuf[r, pl.ds(j * 16, 16)] = zero16
    for r in range(TAIL, BLK):
        for j in range(D // 16):
            tbuf[r, pl.ds(j * 16, 16)] = zero16

    iota16 = lax.iota(jnp.int32, 16)
    for k in range(NUM_SEGMENTS // BLK):
        for j in range(BLK // 16):
            iibuf[k, pl.ds(j * 16, 16)] = iota16 + (k * BLK + j * 16)

    def zero_pacc(r, carry):
        for j in range(D // 16):
            pacc[r, pl.ds(j * 16, 16)] = zero16
        return carry

    lax.fori_loop(0, NUM_SEGMENTS, zero_pacc, None)

    pltpu.sync_copy(zbuf, acc.at[pl.ds(s * SEGS_PER_TILE, SEGS_PER_TILE)])
    plsc.subcore_barrier()

    def start_block(i, b):
        pltpu.async_copy(
            feat.at[pl.ds(base + i * BLK, BLK), :], fbuf.at[b], sems[b])

    def wait_block(b):
        pltpu.make_async_copy(
            feat.at[pl.ds(0, BLK), :], fbuf.at[b], sems[b]).wait()

    start_block(0, 0)
    start_block(1, 1)

    pltpu.make_async_copy(
        ids2.at[pl.ds(0, IDROWS)], ibuf, semi).wait()
    pltpu.make_async_copy(
        feat.at[pl.ds(0, TAIL), :], tbuf.at[pl.ds(0, TAIL)], semi).wait()

    def loop_body(iv, carry):
        for b in range(2):
            i = 2 * iv + b
            wait_block(b)

            m = ibuf[i, pl.ds(0, 16)][0]
            mx = ibuf[i, pl.ds(BLK - 16, 16)][15]

            @pl.when(m == mx)
            def _dense():
                def row_body(it, regs):
                    new = regs
                    for u in range(RUNROLL):
                        r = it * RUNROLL + u
                        new = tuple(
                            new[j] + fbuf[b, r, pl.ds(j * 16, 16)]
                            for j in range(D // 16))
                    return new

                regs = lax.fori_loop(
                    0, BLK // RUNROLL, row_body,
                    tuple(jnp.zeros((16,), jnp.float32)
                          for _ in range(D // 16)))
                for j in range(D // 16):
                    pacc[m, pl.ds(j * 16, 16)] = (
                        pacc[m, pl.ds(j * 16, 16)] + regs[j])

            @pl.when(m != mx)
            def _mixed():
                pltpu.sync_copy(fbuf.at[b], acc.at[ibuf.at[i]], add=True)

            @pl.when(i + 2 < NFULL)
            def _prefetch():
                start_block(i + 2, b)
        return carry

    lax.fori_loop(0, NFULL // 2, loop_body, None)

    pltpu.sync_copy(tbuf, acc.at[ibuf.at[NFULL]], add=True)

    for k in range(NUM_SEGMENTS // BLK):
        pltpu.sync_copy(
            pacc.at[pl.ds(k * BLK, BLK)], acc.at[iibuf.at[k]], add=True)

    plsc.subcore_barrier()
    seg0 = s * SEGS_PER_TILE
    pltpu.sync_copy(
        acc.at[pl.ds(seg0, SEGS_PER_TILE)],
        out.at[c, pl.ds(seg0, SEGS_PER_TILE), :])


_seg_sum = pl.kernel(
    _seg_sum_body,
    out_type=jax.ShapeDtypeStruct((NC, NUM_SEGMENTS, D), jnp.float32),
    mesh=_mesh,
    scratch_types=[
        pltpu.VMEM((2, BLK, D), jnp.float32),
        pltpu.VMEM((BLK, D), jnp.float32),
        pltpu.VMEM((IDROWS, BLK), jnp.int32),
        pltpu.VMEM((NUM_SEGMENTS // BLK, BLK), jnp.int32),
        pltpu.VMEM((SEGS_PER_TILE, D), jnp.float32),
        pltpu.VMEM((NUM_SEGMENTS, D), jnp.float32),
        pltpu.VMEM_SHARED((NUM_SEGMENTS, D), jnp.float32),
        pltpu.SemaphoreType.DMA,
        pltpu.SemaphoreType.DMA,
        pltpu.SemaphoreType.DMA,
    ],
)


def _combine_body(p_ref, o_ref):
    o_ref[...] = p_ref[0] + p_ref[1]


_combine = pl.pallas_call(
    _combine_body,
    out_shape=jax.ShapeDtypeStruct((NUM_SEGMENTS, D), jnp.float32),
)


def kernel(feat, segment_ids):
    ids2 = jnp.pad(
        segment_ids.reshape(NW, ROWS_PER_TILE),
        ((0, 0), (0, IDROWS * BLK - ROWS_PER_TILE)),
    ).reshape(NW * IDROWS, BLK)
    partials = _seg_sum(feat, ids2)
    return _combine(partials)
